# Initial kernel scaffold; baseline (speedup 1.0000x reference)
import jax
import jax.numpy as jnp
from jax.experimental import pallas as pl


def kernel(x, nearest_nodes, nearest_dists, W, b):
    raise NotImplementedError("write your pallas kernel here")


# speedup vs baseline: 5.0000x; 5.0000x over previous
import functools

import jax
import jax.numpy as jnp
from jax import lax
from jax.experimental import pallas as pl
from jax.experimental.pallas import tpu as pltpu
from jax.experimental.pallas import tpu_sc as plsc

H = 4
K1 = 17
CN = 64
ED = CN * K1
GSZ = 64
NG = ED // GSZ
SIGMA_RATIO = 400.0
BETA = 0.8
THR = 1e-5


def _r_kernel(nd_ref, r_ref):
    d = nd_ref[...]
    m = jnp.max(d)
    s = -(SIGMA_RATIO * SIGMA_RATIO * 0.25) / (m * m)
    r_ref[...] = jnp.exp(d * d * s)


def _mm_kernel(a_ref, w_ref, b_ref, o_ref):
    y = jnp.dot(a_ref[...], w_ref[...], preferred_element_type=jnp.float32)
    y = y + b_ref[...]
    o_ref[...] = y * jax.nn.sigmoid(BETA * y)


def _make_sc_agg(nch):
    info = plsc.get_sparse_core_info()
    nw = info.num_cores * info.num_subcores
    chunks_per_worker = nch // nw
    mesh = plsc.VectorSubcoreMesh(core_axis_name="c", subcore_axis_name="s")

    @functools.partial(
        pl.kernel,
        mesh=mesh,
        compiler_params=pltpu.CompilerParams(
            needs_layout_passes=False, use_tc_tiling_on_sc=False
        ),
        out_type=jax.ShapeDtypeStruct((nch, CN, H * 32), jnp.float32),
        scratch_types=[
            pltpu.VMEM((ED,), jnp.int32),
            pltpu.VMEM((ED, 16), jnp.float32),
            pltpu.VMEM((ED, 32), jnp.float32),
            pltpu.VMEM((CN, H * 32), jnp.float32),
            pltpu.SemaphoreType.DMA,
        ],
    )
    def k(xt_hbm, nn_hbm, r_hbm, out_hbm, idx_v, r_v, rows_v, out_v, sem):
        wid = lax.axis_index("s") * info.num_cores + lax.axis_index("c")

        def do_chunk(c):
            pltpu.sync_copy(nn_hbm.at[c], idx_v)
            pltpu.sync_copy(r_hbm.at[c], r_v)
            pltpu.async_copy(xt_hbm.at[idx_v], rows_v, sem).wait()

            def node_body(n, carry):
                e0 = n * K1
                acc = [jnp.zeros((16,), jnp.float32) for _ in range(2 * H)]
                for kk in range(K1):
                    e = e0 + kk
                    rb = r_v[e, pl.ds(0, 16)]
                    row0 = rows_v[e, pl.ds(0, 16)]
                    row1 = rows_v[e, pl.ds(16, 16)]
                    p = rb
                    for h in range(H):
                        w = jnp.where(p < THR, 0.0, p)
                        acc[2 * h] = acc[2 * h] + w * row0
                        acc[2 * h + 1] = acc[2 * h + 1] + w * row1
                        if h + 1 < H:
                            p = p * rb
                for h in range(H):
                    out_v[n, pl.ds(h * 32, 16)] = acc[2 * h]
                    out_v[n, pl.ds(h * 32 + 16, 16)] = acc[2 * h + 1]
                return carry

            lax.fori_loop(0, CN, node_body, jnp.int32(0))
            pltpu.sync_copy(out_v, out_hbm.at[c])

        for i in range(chunks_per_worker):
            do_chunk(wid + i * nw)

    return k


def kernel(x, nearest_nodes, nearest_dists, W, b):
    _, t_in, n_nodes, c_feat = x.shape
    nch = -(-n_nodes // CN)
    nch = -(-nch // 32) * 32
    n_pad = nch * CN

    xt = jnp.transpose(x[0], (1, 0, 2)).reshape(n_nodes, t_in * c_feat)
    xt = jnp.pad(xt, ((0, 0), (0, 32 - t_in * c_feat)))

    r = pl.pallas_call(
        _r_kernel,
        out_shape=jax.ShapeDtypeStruct(nearest_dists.shape, jnp.float32),
    )(nearest_dists)

    pad_e = n_pad * K1 - n_nodes * K1
    nn_p = jnp.pad(nearest_nodes.reshape(-1), (0, pad_e)).reshape(nch, ED)
    r_p = jnp.pad(r.reshape(-1), (0, pad_e)).reshape(nch, ED)
    r16_p = jnp.broadcast_to(r_p[:, :, None], (nch, ED, 16))

    agg = _make_sc_agg(nch)(xt, nn_p, r16_p)
    A = agg.reshape(n_pad, H * 32)

    Wr = W.reshape(t_in, t_in, H)
    Wt = jnp.transpose(Wr, (2, 1, 0))
    Z = Wt[:, :, None, None, :] * jnp.eye(c_feat, dtype=W.dtype)[None, None, :, :, None]
    Wcore = Z.reshape(H, t_in * c_feat, c_feat * t_in)
    Wbig = jnp.pad(Wcore, ((0, 0), (0, 32 - t_in * c_feat), (0, 0))).reshape(H * 32, c_feat * t_in)
    Bbig = jnp.concatenate([b, b]).reshape(1, c_feat * t_in)

    mblk = n_pad // 8
    Y2 = pl.pallas_call(
        _mm_kernel,
        grid=(8,),
        in_specs=[
            pl.BlockSpec((mblk, H * 32), lambda i: (i, 0)),
            pl.BlockSpec((H * 32, c_feat * t_in), lambda i: (0, 0)),
            pl.BlockSpec((1, c_feat * t_in), lambda i: (0, 0)),
        ],
        out_specs=pl.BlockSpec((mblk, c_feat * t_in), lambda i: (i, 0)),
        out_shape=jax.ShapeDtypeStruct((n_pad, c_feat * t_in), jnp.float32),
    )(A, Wbig, Bbig)

    y = jnp.transpose(Y2[:n_nodes].reshape(n_nodes, c_feat, t_in), (2, 0, 1))[None]
    return jnp.concatenate([x, y], axis=1)
